# baseline (speedup 1.0000x reference)
import jax
import jax.numpy as jnp
from jax import lax
from jax.experimental import pallas as pl
from jax.experimental.pallas import tpu as pltpu

B = 256
IN_DIM = 512
OUT_DIM = 10000
BJ = 200
NJ = OUT_DIM // BJ


def _gcn_kernel(x_ref, w_ref, adj_ref, bias_ref, out_ref, sT_ref):
    @pl.when(pl.program_id(0) == 0)
    def _():
        s = jnp.dot(
            x_ref[...].astype(jnp.bfloat16),
            w_ref[...].astype(jnp.bfloat16),
            preferred_element_type=jnp.float32,
        ).astype(jnp.bfloat16)
        eye = (
            lax.broadcasted_iota(jnp.int32, (B, B), 0)
            == lax.broadcasted_iota(jnp.int32, (B, B), 1)
        ).astype(jnp.bfloat16)
        sT_ref[...] = lax.dot_general(
            s,
            eye,
            (((0,), (0,)), ((), ())),
            preferred_element_type=jnp.float32,
        ).astype(jnp.bfloat16)

    out_ref[...] = (
        jnp.dot(
            adj_ref[...].astype(jnp.bfloat16),
            sT_ref[...],
            preferred_element_type=jnp.float32,
        )
        + bias_ref[...]
    )


def kernel(input, adj, weight, bias):
    outT = pl.pallas_call(
        _gcn_kernel,
        grid=(NJ,),
        in_specs=[
            pl.BlockSpec((B, IN_DIM), lambda j: (0, 0)),
            pl.BlockSpec((IN_DIM, OUT_DIM), lambda j: (0, 0)),
            pl.BlockSpec((BJ, OUT_DIM), lambda j: (j, 0)),
            pl.BlockSpec((BJ, 1), lambda j: (j, 0)),
        ],
        out_specs=pl.BlockSpec((BJ, B), lambda j: (j, 0)),
        out_shape=jax.ShapeDtypeStruct((OUT_DIM, B), jnp.float32),
        scratch_shapes=[pltpu.VMEM((OUT_DIM, B), jnp.bfloat16)],
    )(input, weight, adj, bias.reshape(OUT_DIM, 1))
    return outT.T


# speedup vs baseline: 1.1102x; 1.1102x over previous
import jax
import jax.numpy as jnp
from jax import lax
from jax.experimental import pallas as pl
from jax.experimental.pallas import tpu as pltpu

B = 256
IN_DIM = 512
OUT_DIM = 10000
BJ = 200
NJ = OUT_DIM // BJ
CH = 2000
NB = OUT_DIM // CH


def _gcn_kernel(wT_ref, x_ref, adj_ref, bias_ref, out_ref, sT_ref):
    j = pl.program_id(0)

    @pl.when(j < NB)
    def _():
        sT_ref[pl.ds(j * CH, CH), :] = lax.dot_general(
            wT_ref[...], x_ref[...],
            (((1,), (1,)), ((), ())),
            preferred_element_type=jnp.float32,
        )

    @pl.when(j >= NB - 1)
    def _():
        out_ref[...] = (
            jnp.dot(adj_ref[...], sT_ref[...], preferred_element_type=jnp.float32)
            + bias_ref[...]
        )


def kernel(input, adj, weight, bias):
    wT = weight.T.astype(jnp.bfloat16)
    x = input.astype(jnp.bfloat16)
    outT = pl.pallas_call(
        _gcn_kernel,
        grid=(NB - 1 + NJ,),
        in_specs=[
            pl.BlockSpec((CH, IN_DIM), lambda j: (jnp.minimum(j, NB - 1), 0)),
            pl.BlockSpec((B, IN_DIM), lambda j: (0, 0)),
            pl.BlockSpec((BJ, OUT_DIM), lambda j: (jnp.maximum(j - (NB - 1), 0), 0)),
            pl.BlockSpec((BJ, 1), lambda j: (jnp.maximum(j - (NB - 1), 0), 0)),
        ],
        out_specs=pl.BlockSpec((BJ, B), lambda j: (jnp.maximum(j - (NB - 1), 0), 0)),
        out_shape=jax.ShapeDtypeStruct((OUT_DIM, B), jnp.float32),
        scratch_shapes=[pltpu.VMEM((OUT_DIM, B), jnp.float32)],
    )(wT, x, adj, bias.reshape(OUT_DIM, 1))
    return outT.T
